# baseline (speedup 1.0000x reference)
import functools

import jax
import jax.numpy as jnp
from jax import lax
from jax.experimental import pallas as pl
from jax.experimental.pallas import tpu as pltpu
from jax.experimental.pallas import tpu_sc as plsc

_NTOKEN = 100000
_NINP = 64
_BATCH = 16384
_HIST = 50

_NC = 2
_NS = 16
_NW = _NC * _NS
_BPW = _BATCH // _NW
_HB = 256
_L = 16


def _make_gather():
    mesh = plsc.VectorSubcoreMesh(core_axis_name="c", subcore_axis_name="s")

    @functools.partial(
        pl.kernel,
        mesh=mesh,
        out_type=jax.ShapeDtypeStruct((_HIST, 8, 128, 8, 128), jnp.float32),
        scratch_types=[
            pltpu.VMEM((_HIST, _BPW), jnp.int32),
            pltpu.VMEM((2, _HB, _NINP), jnp.float32),
            pltpu.VMEM((2, 8, 2, 8, 128), jnp.float32),
            pltpu.SemaphoreType.DMA,
            pltpu.SemaphoreType.DMA,
            pltpu.SemaphoreType.DMA,
            pltpu.SemaphoreType.DMA,
            pltpu.SemaphoreType.DMA,
        ],
        compiler_params=pltpu.CompilerParams(
            use_tc_tiling_on_sc=False, needs_layout_passes=False
        ),
    )
    def gather(table_hbm, idx_hbm, out_hbm, idx_v, grows, tbuf,
               isem, gs0, gs1, ws0, ws1):
        gsem = (gs0, gs1)
        wsem = (ws0, ws1)
        wid = lax.axis_index("s") * _NC + lax.axis_index("c")
        b0 = wid * _BPW
        pltpu.async_copy(idx_hbm.at[:, pl.ds(b0, _BPW)], idx_v, isem).wait()

        def fire_gathers(h, half, p):
            for c in range(2):
                pltpu.async_copy(
                    table_hbm.at[idx_v.at[h, pl.ds(half * _HB + c * 128, 128)]],
                    grows.at[p, pl.ds(c * 128, 128)],
                    gsem[p],
                )

        def drain_gathers(p):
            for _ in range(2):
                pltpu.make_async_copy(
                    table_hbm.at[idx_v.at[0, pl.ds(0, 128)]],
                    grows.at[p, pl.ds(0, 128)],
                    gsem[p],
                ).wait()

        def fire_write(h, half, p):
            pltpu.async_copy(
                tbuf.at[p],
                out_hbm.at[h, :, pl.ds(wid * 4 + half * 2, 2)],
                wsem[p],
            )

        def drain_write(p):
            pltpu.make_async_copy(
                tbuf.at[p],
                out_hbm.at[0, :, pl.ds(0, 2)],
                wsem[p],
            ).wait()

        iota = lax.iota(jnp.int32, _L)

        def transpose(p):
            for bh in range(2):

                def tbody(b8, carry):
                    rows = iota + (bh * 128 + b8 * _L)
                    c0 = b8 * _L
                    for j in range(_NINP):
                        cols = jnp.full((_L,), j, jnp.int32)
                        v = plsc.load_gather(grows.at[p], [rows, cols])
                        tbuf[p, j // 8, bh, j % 8, pl.ds(c0, _L)] = v
                    return carry

                lax.fori_loop(0, 8, tbody, 0)

        fire_gathers(0, 0, 0)

        def body(k, carry):
            fire_gathers(k, 1, 1)
            drain_gathers(0)

            @pl.when(k > 0)
            def _():
                drain_write(0)

            transpose(0)
            fire_write(k, 0, 0)

            @pl.when(k < _HIST - 1)
            def _():
                fire_gathers(k + 1, 0, 0)

            drain_gathers(1)

            @pl.when(k > 0)
            def _():
                drain_write(1)

            transpose(1)
            fire_write(k, 1, 1)
            return carry

        lax.fori_loop(0, _HIST, body, 0)
        drain_write(0)
        drain_write(1)

    return gather


_gather = _make_gather()


def kernel(input, word_embed_weight):
    idx2 = input.T
    out5 = _gather(word_embed_weight, idx2)
    return out5.transpose(2, 4, 0, 1, 3).reshape(_BATCH, _HIST, _NINP)
